# baseline (speedup 1.0000x reference)
import functools

import jax
import jax.numpy as jnp
from jax import lax
from jax.experimental import pallas as pl
from jax.experimental.pallas import tpu as pltpu
from jax.experimental.pallas import tpu_sc as plsc

N_NODES = 10000
N_EDGES = 320000
H = 128
L = 16
CHUNK = 40
N_CHUNKS = N_EDGES // CHUNK
NW = 32
MY_CHUNKS = N_CHUNKS // NW
LOOP_CHUNKS = (MY_CHUNKS // 4) * 4
SLAB = 624

_sc_mesh = plsc.VectorSubcoreMesh(core_axis_name="c", subcore_axis_name="s")


@functools.partial(
    pl.kernel,
    out_type=jax.ShapeDtypeStruct((2 * N_NODES, H), jnp.float32),
    mesh=_sc_mesh,
    scratch_types=[
        [pltpu.VMEM((CHUNK,), jnp.int32) for _ in range(4)],
        [pltpu.VMEM((CHUNK,), jnp.int32) for _ in range(4)],
        [pltpu.VMEM((CHUNK, H), jnp.float32) for _ in range(4)],
        [pltpu.VMEM((CHUNK, H), jnp.float32) for _ in range(4)],
        pltpu.VMEM_SHARED((N_NODES, H), jnp.float32),
        [pltpu.SemaphoreType.DMA for _ in range(4)],
        [pltpu.SemaphoreType.DMA for _ in range(4)],
        [pltpu.SemaphoreType.DMA for _ in range(4)],
        [pltpu.SemaphoreType.DMA for _ in range(4)],
    ],
)
def _sc_aggregate(x_hbm, src_hbm, dst_hbm, ea_hbm, out_hbm,
                  svs, dvs, xrs, eas, acc_sh, gss, ess, iss, sss):
    c = lax.axis_index("c")
    s = lax.axis_index("s")
    wid = s * 2 + c

    zeros16 = jnp.zeros((L,), jnp.float32)
    zb = eas[0]

    def zrow(r, _):
        for j in range(H // L):
            zb[r, pl.ds(j * L, L)] = zeros16
        return 0

    lax.fori_loop(0, CHUNK, zrow, 0)
    slab = s * SLAB
    for i in range(SLAB // CHUNK):
        pltpu.sync_copy(zb, acc_sh.at[pl.ds(slab + i * CHUNK, CHUNK), :])
    pltpu.sync_copy(zb.at[pl.ds(0, SLAB % CHUNK), :],
                    acc_sh.at[pl.ds(slab + (SLAB // CHUNK) * CHUNK,
                                    SLAB % CHUNK), :])

    @pl.when(s == 15)
    def _():
        pltpu.sync_copy(zb.at[pl.ds(0, 16), :],
                        acc_sh.at[pl.ds(16 * SLAB, 16), :])

    plsc.subcore_barrier()

    ebase = wid * MY_CHUNKS * CHUNK

    def idx_start(kk, b):
        off = ebase + kk * CHUNK
        pltpu.async_copy(src_hbm.at[pl.ds(off, CHUNK)], svs[b], iss[b])
        pltpu.async_copy(dst_hbm.at[pl.ds(off, CHUNK)], dvs[b], iss[b])

    def idx_wait(b):
        pltpu.make_async_copy(src_hbm.at[pl.ds(0, CHUNK)], svs[b],
                              iss[b]).wait()
        pltpu.make_async_copy(dst_hbm.at[pl.ds(0, CHUNK)], dvs[b],
                              iss[b]).wait()

    def data_start(kk, b):
        pltpu.async_copy(x_hbm.at[svs[b]], xrs[b], gss[b])
        pltpu.async_copy(ea_hbm.at[pl.ds(ebase + kk * CHUNK, CHUNK), :],
                         eas[b], ess[b])

    def data_wait(b):
        pltpu.make_async_copy(x_hbm.at[svs[b]], xrs[b], gss[b]).wait()
        pltpu.make_async_copy(ea_hbm.at[pl.ds(0, CHUNK), :], eas[b],
                              ess[b]).wait()

    def compute(b):
        xr, ea = xrs[b], eas[b]

        @plsc.parallel_loop(0, CHUNK, unroll=4)
        def _(r):
            for j in range(H // L):
                sl = pl.ds(j * L, L)
                ea[r, sl] = jnp.maximum(xr[r, sl] + ea[r, sl], 0.0)

    def scatter_start(b):
        pltpu.async_copy(eas[b], acc_sh.at[dvs[b]], sss[b], add=True)

    def scatter_wait(b):
        pltpu.make_async_copy(eas[b], acc_sh.at[dvs[b]], sss[b]).wait()

    def stage(kk, b, g2, g3):
        b2 = (b + 2) % 4
        b3 = (b + 3) % 4

        def prefetch():
            idx_wait(b2)
            data_start(kk + 2, b2)

        def nextidx():
            idx_start(kk + 3, b3)

        def retire():
            scatter_wait(b3)

        if g2 is True:
            prefetch()
        elif g2 is not False:
            pl.when(g2)(prefetch)

        data_wait(b)
        compute(b)

        if isinstance(kk, int):
            if kk > 0:
                retire()
        else:
            pl.when(kk > 0)(retire)

        if g3 is True:
            nextidx()
        elif g3 is not False:
            pl.when(g3)(nextidx)

        scatter_start(b)

    idx_start(0, 0)
    idx_start(1, 1)
    idx_wait(0)
    data_start(0, 0)
    idx_wait(1)
    data_start(1, 1)
    idx_start(2, 2)

    @pl.loop(0, LOOP_CHUNKS, step=4)
    def _(k):
        for b in range(4):
            kk = k + b
            stage(kk, b, kk + 2 < MY_CHUNKS, kk + 3 < MY_CHUNKS)

    stage(LOOP_CHUNKS, 0, False, False)
    stage(LOOP_CHUNKS + 1, 1, False, False)
    scatter_wait(1)

    plsc.subcore_barrier()

    obase = c * N_NODES + slab
    for i in range(SLAB // CHUNK):
        pltpu.sync_copy(acc_sh.at[pl.ds(slab + i * CHUNK, CHUNK), :],
                        out_hbm.at[pl.ds(obase + i * CHUNK, CHUNK), :])
    pltpu.sync_copy(
        acc_sh.at[pl.ds(slab + (SLAB // CHUNK) * CHUNK, SLAB % CHUNK), :],
        out_hbm.at[pl.ds(obase + (SLAB // CHUNK) * CHUNK, SLAB % CHUNK), :])

    @pl.when(s == 15)
    def _():
        pltpu.sync_copy(acc_sh.at[pl.ds(16 * SLAB, 16), :],
                        out_hbm.at[pl.ds(c * N_NODES + 16 * SLAB, 16), :])


def _tc_layer_body(x_ref, p0_ref, p1_ref, wt_ref, b_ref, o_ref):
    h = x_ref[...] + p0_ref[...] + p1_ref[...]
    y = jnp.dot(h, wt_ref[...], preferred_element_type=jnp.float32) + b_ref[...]
    o_ref[...] = jnp.maximum(y, 0.0)


_TC_BLOCK = 1000


def _tc_layer(x, p0, p1, wt, b2d):
    grid = (N_NODES // _TC_BLOCK,)
    return pl.pallas_call(
        _tc_layer_body,
        grid=grid,
        in_specs=[
            pl.BlockSpec((_TC_BLOCK, H), lambda i: (i, 0)),
            pl.BlockSpec((_TC_BLOCK, H), lambda i: (i, 0)),
            pl.BlockSpec((_TC_BLOCK, H), lambda i: (i, 0)),
            pl.BlockSpec((H, H), lambda i: (0, 0)),
            pl.BlockSpec((1, H), lambda i: (0, 0)),
        ],
        out_specs=pl.BlockSpec((_TC_BLOCK, H), lambda i: (i, 0)),
        out_shape=jax.ShapeDtypeStruct((N_NODES, H), jnp.float32),
    )(x, p0, p1, wt, b2d)


def kernel(node_feats, edge_index, edge_attrs, W1, b1, W2, b2):
    src = edge_index[0].astype(jnp.int32)
    dst = edge_index[1].astype(jnp.int32)
    x = node_feats

    p = _sc_aggregate(x, src, dst, edge_attrs)
    x1 = _tc_layer(x, p[:N_NODES], p[N_NODES:], W1.T, b1.reshape(1, H))
    q = _sc_aggregate(x1, src, dst, edge_attrs)
    x2 = _tc_layer(x1, q[:N_NODES], q[N_NODES:], W2.T, b2.reshape(1, H))
    return x2


# speedup vs baseline: 1.1714x; 1.1714x over previous
import functools

import jax
import jax.numpy as jnp
from jax import lax
from jax.experimental import pallas as pl
from jax.experimental.pallas import tpu as pltpu
from jax.experimental.pallas import tpu_sc as plsc

N_NODES = 10000
N_EDGES = 320000
H = 128
L = 16
CHUNK = 64
N_CHUNKS = N_EDGES // CHUNK
NW = 32
BASE_CHUNKS = N_CHUNKS // NW
EXTRA = N_CHUNKS - BASE_CHUNKS * NW
SLAB = 624

_sc_mesh = plsc.VectorSubcoreMesh(core_axis_name="c", subcore_axis_name="s")


@functools.partial(
    pl.kernel,
    out_type=jax.ShapeDtypeStruct((2 * N_NODES, H), jnp.float32),
    mesh=_sc_mesh,
    scratch_types=[
        [pltpu.VMEM((CHUNK,), jnp.int32) for _ in range(6)],
        [pltpu.VMEM((CHUNK,), jnp.int32) for _ in range(6)],
        [pltpu.VMEM((CHUNK, H), jnp.float32) for _ in range(3)],
        [pltpu.VMEM((CHUNK, H), jnp.float32) for _ in range(3)],
        pltpu.VMEM_SHARED((N_NODES, H), jnp.float32),
        [pltpu.SemaphoreType.DMA for _ in range(3)],
        [pltpu.SemaphoreType.DMA for _ in range(3)],
        [pltpu.SemaphoreType.DMA for _ in range(6)],
        [pltpu.SemaphoreType.DMA for _ in range(3)],
    ],
)
def _sc_aggregate(x_hbm, src_hbm, dst_hbm, ea_hbm, out_hbm,
                  svs, dvs, xrs, eas, acc_sh, gss, ess, iss, sss):
    c = lax.axis_index("c")
    s = lax.axis_index("s")
    wid = s * 2 + c

    zeros16 = jnp.zeros((L,), jnp.float32)
    zb = eas[0]

    def zrow(r, _):
        for j in range(H // L):
            zb[r, pl.ds(j * L, L)] = zeros16
        return 0

    lax.fori_loop(0, CHUNK, zrow, 0)
    slab = s * SLAB
    for i in range(SLAB // CHUNK):
        pltpu.sync_copy(zb, acc_sh.at[pl.ds(slab + i * CHUNK, CHUNK), :])
    pltpu.sync_copy(zb.at[pl.ds(0, SLAB % CHUNK), :],
                    acc_sh.at[pl.ds(slab + (SLAB // CHUNK) * CHUNK,
                                    SLAB % CHUNK), :])

    @pl.when(s == 15)
    def _():
        pltpu.sync_copy(zb.at[pl.ds(0, 16), :],
                        acc_sh.at[pl.ds(16 * SLAB, 16), :])

    plsc.subcore_barrier()

    n_my = BASE_CHUNKS + jnp.where(wid < EXTRA, 1, 0)
    ebase = (BASE_CHUNKS * wid + jnp.minimum(wid, EXTRA)) * CHUNK

    def idx_start(kk, si):
        off = ebase + kk * CHUNK
        pltpu.async_copy(src_hbm.at[pl.ds(off, CHUNK)], svs[si], iss[si])
        pltpu.async_copy(dst_hbm.at[pl.ds(off, CHUNK)], dvs[si], iss[si])

    def idx_wait(si):
        pltpu.make_async_copy(src_hbm.at[pl.ds(0, CHUNK)], svs[si],
                              iss[si]).wait()
        pltpu.make_async_copy(dst_hbm.at[pl.ds(0, CHUNK)], dvs[si],
                              iss[si]).wait()

    def data_start(kk, b, si):
        pltpu.async_copy(x_hbm.at[svs[si]], xrs[b], gss[b])
        pltpu.async_copy(ea_hbm.at[pl.ds(ebase + kk * CHUNK, CHUNK), :],
                         eas[b], ess[b])

    def data_wait(b, si):
        pltpu.make_async_copy(x_hbm.at[svs[si]], xrs[b], gss[b]).wait()
        pltpu.make_async_copy(ea_hbm.at[pl.ds(0, CHUNK), :], eas[b],
                              ess[b]).wait()

    def compute(b):
        xr, ea = xrs[b], eas[b]

        @plsc.parallel_loop(0, CHUNK, unroll=4)
        def _(r):
            for j in range(H // L):
                sl = pl.ds(j * L, L)
                ea[r, sl] = jnp.maximum(xr[r, sl] + ea[r, sl], 0.0)

    def scatter_start(b, si):
        pltpu.async_copy(eas[b], acc_sh.at[dvs[si]], sss[b], add=True)

    def scatter_wait(b, si):
        pltpu.make_async_copy(eas[b], acc_sh.at[dvs[si]], sss[b]).wait()

    for kk in range(4):
        idx_start(kk, kk)
    idx_wait(0)
    data_start(0, 0, 0)

    @pl.loop(0, BASE_CHUNKS, step=6)
    def _(k):
        for u in range(6):
            kk = k + u
            b = u % 3
            si = u % 6
            bn = (u + 1) % 3
            sin = (u + 1) % 6
            bp = (u + 2) % 3
            sip = (u + 5) % 6

            @pl.when(kk + 1 < n_my)
            def _():
                idx_wait(sin)
                data_start(kk + 1, bn, sin)

            data_wait(b, si)
            compute(b)
            scatter_start(b, si)

            @pl.when(kk > 0)
            def _():
                scatter_wait(bp, sip)

            @pl.when(kk + 4 < n_my)
            def _():
                idx_start(kk + 4, (u + 4) % 6)

    @pl.when(wid < EXTRA)
    def _():
        data_wait(0, 0)
        compute(0)
        scatter_start(0, 0)
        scatter_wait(2, 5)
        scatter_wait(0, 0)

    @pl.when(wid >= EXTRA)
    def _():
        scatter_wait(2, 5)

    plsc.subcore_barrier()

    obase = c * N_NODES + slab
    for i in range(SLAB // CHUNK):
        pltpu.sync_copy(acc_sh.at[pl.ds(slab + i * CHUNK, CHUNK), :],
                        out_hbm.at[pl.ds(obase + i * CHUNK, CHUNK), :])
    pltpu.sync_copy(
        acc_sh.at[pl.ds(slab + (SLAB // CHUNK) * CHUNK, SLAB % CHUNK), :],
        out_hbm.at[pl.ds(obase + (SLAB // CHUNK) * CHUNK, SLAB % CHUNK), :])

    @pl.when(s == 15)
    def _():
        pltpu.sync_copy(acc_sh.at[pl.ds(16 * SLAB, 16), :],
                        out_hbm.at[pl.ds(c * N_NODES + 16 * SLAB, 16), :])


def _tc_layer_body(x_ref, p0_ref, p1_ref, wt_ref, b_ref, o_ref):
    h = x_ref[...] + p0_ref[...] + p1_ref[...]
    y = jnp.dot(h, wt_ref[...], preferred_element_type=jnp.float32) + b_ref[...]
    o_ref[...] = jnp.maximum(y, 0.0)


_TC_BLOCK = 1000


def _tc_layer(x, p0, p1, wt, b2d):
    grid = (N_NODES // _TC_BLOCK,)
    return pl.pallas_call(
        _tc_layer_body,
        grid=grid,
        in_specs=[
            pl.BlockSpec((_TC_BLOCK, H), lambda i: (i, 0)),
            pl.BlockSpec((_TC_BLOCK, H), lambda i: (i, 0)),
            pl.BlockSpec((_TC_BLOCK, H), lambda i: (i, 0)),
            pl.BlockSpec((H, H), lambda i: (0, 0)),
            pl.BlockSpec((1, H), lambda i: (0, 0)),
        ],
        out_specs=pl.BlockSpec((_TC_BLOCK, H), lambda i: (i, 0)),
        out_shape=jax.ShapeDtypeStruct((N_NODES, H), jnp.float32),
    )(x, p0, p1, wt, b2d)


def kernel(node_feats, edge_index, edge_attrs, W1, b1, W2, b2):
    src = edge_index[0].astype(jnp.int32)
    dst = edge_index[1].astype(jnp.int32)
    x = node_feats

    p = _sc_aggregate(x, src, dst, edge_attrs)
    x1 = _tc_layer(x, p[:N_NODES], p[N_NODES:], W1.T, b1.reshape(1, H))
    q = _sc_aggregate(x1, src, dst, edge_attrs)
    x2 = _tc_layer(x1, q[:N_NODES], q[N_NODES:], W2.T, b2.reshape(1, H))
    return x2


# speedup vs baseline: 1.2057x; 1.0293x over previous
import functools

import jax
import jax.numpy as jnp
from jax import lax
from jax.experimental import pallas as pl
from jax.experimental.pallas import tpu as pltpu
from jax.experimental.pallas import tpu_sc as plsc

N_NODES = 10000
N_EDGES = 320000
H = 128
L = 16
CHUNK = 64
N_CHUNKS = N_EDGES // CHUNK
NW = 32
BASE_CHUNKS = N_CHUNKS // NW
EXTRA = N_CHUNKS - BASE_CHUNKS * NW
SLAB = 624

_sc_mesh = plsc.VectorSubcoreMesh(core_axis_name="c", subcore_axis_name="s")


@functools.partial(
    pl.kernel,
    out_type=jax.ShapeDtypeStruct((2 * N_NODES, H), jnp.float32),
    mesh=_sc_mesh,
    scratch_types=[
        [pltpu.VMEM((CHUNK,), jnp.int32) for _ in range(6)],
        [pltpu.VMEM((CHUNK,), jnp.int32) for _ in range(6)],
        [pltpu.VMEM((CHUNK, H), jnp.float32) for _ in range(3)],
        [pltpu.VMEM((CHUNK, H), jnp.float32) for _ in range(3)],
        pltpu.VMEM_SHARED((N_NODES, H), jnp.float32),
        [pltpu.SemaphoreType.DMA for _ in range(3)],
        [pltpu.SemaphoreType.DMA for _ in range(3)],
        [pltpu.SemaphoreType.DMA for _ in range(6)],
        [pltpu.SemaphoreType.DMA for _ in range(3)],
    ],
)
def _sc_aggregate(x_hbm, src_hbm, dst_hbm, ea_hbm, out_hbm,
                  svs, dvs, xrs, eas, acc_sh, gss, ess, iss, sss):
    c = lax.axis_index("c")
    s = lax.axis_index("s")
    wid = s * 2 + c

    zeros16 = jnp.zeros((L,), jnp.float32)
    zb = eas[0]

    def zrow(r, _):
        for j in range(H // L):
            zb[r, pl.ds(j * L, L)] = zeros16
        return 0

    lax.fori_loop(0, CHUNK, zrow, 0)
    slab = s * SLAB
    for i in range(SLAB // CHUNK):
        pltpu.sync_copy(zb, acc_sh.at[pl.ds(slab + i * CHUNK, CHUNK), :])
    pltpu.sync_copy(zb.at[pl.ds(0, SLAB % CHUNK), :],
                    acc_sh.at[pl.ds(slab + (SLAB // CHUNK) * CHUNK,
                                    SLAB % CHUNK), :])

    @pl.when(s == 15)
    def _():
        pltpu.sync_copy(zb.at[pl.ds(0, 16), :],
                        acc_sh.at[pl.ds(16 * SLAB, 16), :])

    plsc.subcore_barrier()

    n_my = BASE_CHUNKS + jnp.where(wid < EXTRA, 1, 0)
    ebase = (BASE_CHUNKS * wid + jnp.minimum(wid, EXTRA)) * CHUNK

    def idx_start(kk, si):
        off = ebase + kk * CHUNK
        pltpu.async_copy(src_hbm.at[pl.ds(off, CHUNK)], svs[si], iss[si])
        pltpu.async_copy(dst_hbm.at[pl.ds(off, CHUNK)], dvs[si], iss[si])

    def idx_wait(si):
        pltpu.make_async_copy(src_hbm.at[pl.ds(0, CHUNK)], svs[si],
                              iss[si]).wait()
        pltpu.make_async_copy(dst_hbm.at[pl.ds(0, CHUNK)], dvs[si],
                              iss[si]).wait()

    def data_start(kk, b, si):
        pltpu.async_copy(x_hbm.at[svs[si]], xrs[b], gss[b])
        pltpu.async_copy(ea_hbm.at[pl.ds(ebase + kk * CHUNK, CHUNK), :],
                         eas[b], ess[b])

    def data_wait(b, si):
        pltpu.make_async_copy(x_hbm.at[svs[si]], xrs[b], gss[b]).wait()
        pltpu.make_async_copy(ea_hbm.at[pl.ds(0, CHUNK), :], eas[b],
                              ess[b]).wait()

    def compute(b):
        xr, ea = xrs[b], eas[b]

        @plsc.parallel_loop(0, CHUNK, unroll=4)
        def _(r):
            for j in range(H // L):
                sl = pl.ds(j * L, L)
                ea[r, sl] = jnp.maximum(xr[r, sl] + ea[r, sl], 0.0)

    def scatter_start(b, si):
        pltpu.async_copy(eas[b], acc_sh.at[dvs[si]], sss[b], add=True)

    def scatter_wait(b, si):
        pltpu.make_async_copy(eas[b], acc_sh.at[dvs[si]], sss[b]).wait()

    for kk in range(4):
        idx_start(kk, kk)
    idx_wait(0)
    data_start(0, 0, 0)

    @pl.loop(0, BASE_CHUNKS, step=6)
    def _(k):
        for u in range(6):
            kk = k + u
            b = u % 3
            si = u % 6
            bn = (u + 1) % 3
            sin = (u + 1) % 6
            bp = (u + 2) % 3
            sip = (u + 5) % 6

            @pl.when(kk + 1 < n_my)
            def _():
                idx_wait(sin)
                data_start(kk + 1, bn, sin)

            data_wait(b, si)
            compute(b)
            scatter_start(b, si)

            @pl.when(kk > 0)
            def _():
                scatter_wait(bp, sip)

            @pl.when(kk + 4 < n_my)
            def _():
                idx_start(kk + 4, (u + 4) % 6)

    @pl.when(wid < EXTRA)
    def _():
        data_wait(0, 0)
        compute(0)
        scatter_start(0, 0)
        scatter_wait(2, 5)
        scatter_wait(0, 0)

    @pl.when(wid >= EXTRA)
    def _():
        scatter_wait(2, 5)

    plsc.subcore_barrier()

    obase = c * N_NODES + slab
    for i in range(SLAB // CHUNK):
        pltpu.sync_copy(acc_sh.at[pl.ds(slab + i * CHUNK, CHUNK), :],
                        out_hbm.at[pl.ds(obase + i * CHUNK, CHUNK), :])
    pltpu.sync_copy(
        acc_sh.at[pl.ds(slab + (SLAB // CHUNK) * CHUNK, SLAB % CHUNK), :],
        out_hbm.at[pl.ds(obase + (SLAB // CHUNK) * CHUNK, SLAB % CHUNK), :])

    @pl.when(s == 15)
    def _():
        pltpu.sync_copy(acc_sh.at[pl.ds(16 * SLAB, 16), :],
                        out_hbm.at[pl.ds(c * N_NODES + 16 * SLAB, 16), :])


def _tc_layer_body(x_ref, p0_ref, p1_ref, wt_ref, b_ref, o_ref):
    h = x_ref[...] + p0_ref[...] + p1_ref[...]
    y = jnp.dot(h, wt_ref[...], preferred_element_type=jnp.float32) + b_ref[...]
    o_ref[...] = jnp.maximum(y, 0.0)


_TC_BLOCK = 1000


_P_BLOCKS = N_NODES // _TC_BLOCK


def _tc_layer(x, p, wt, b2d):
    grid = (N_NODES // _TC_BLOCK,)
    return pl.pallas_call(
        _tc_layer_body,
        grid=grid,
        in_specs=[
            pl.BlockSpec((_TC_BLOCK, H), lambda i: (i, 0)),
            pl.BlockSpec((_TC_BLOCK, H), lambda i: (i, 0)),
            pl.BlockSpec((_TC_BLOCK, H), lambda i: (i + _P_BLOCKS, 0)),
            pl.BlockSpec((H, H), lambda i: (0, 0)),
            pl.BlockSpec((1, H), lambda i: (0, 0)),
        ],
        out_specs=pl.BlockSpec((_TC_BLOCK, H), lambda i: (i, 0)),
        out_shape=jax.ShapeDtypeStruct((N_NODES, H), jnp.float32),
    )(x, p, p, wt, b2d)


def kernel(node_feats, edge_index, edge_attrs, W1, b1, W2, b2):
    src = edge_index[0].astype(jnp.int32)
    dst = edge_index[1].astype(jnp.int32)
    x = node_feats

    p = _sc_aggregate(x, src, dst, edge_attrs)
    x1 = _tc_layer(x, p, W1.T, b1.reshape(1, H))
    q = _sc_aggregate(x1, src, dst, edge_attrs)
    x2 = _tc_layer(x1, q, W2.T, b2.reshape(1, H))
    return x2


# speedup vs baseline: 1.2074x; 1.0014x over previous
import functools

import jax
import jax.numpy as jnp
from jax import lax
from jax.experimental import pallas as pl
from jax.experimental.pallas import tpu as pltpu
from jax.experimental.pallas import tpu_sc as plsc

N_NODES = 10000
N_EDGES = 320000
H = 128
L = 16
CHUNK = 64
N_CHUNKS = N_EDGES // CHUNK
NW = 32
BASE_CHUNKS = N_CHUNKS // NW
EXTRA = N_CHUNKS - BASE_CHUNKS * NW
SLAB = 624

_sc_mesh = plsc.VectorSubcoreMesh(core_axis_name="c", subcore_axis_name="s")


@functools.partial(
    pl.kernel,
    out_type=jax.ShapeDtypeStruct((2 * N_NODES, H), jnp.float32),
    mesh=_sc_mesh,
    scratch_types=[
        [pltpu.VMEM((CHUNK,), jnp.int32) for _ in range(6)],
        [pltpu.VMEM((CHUNK,), jnp.int32) for _ in range(6)],
        [pltpu.VMEM((CHUNK, H), jnp.float32) for _ in range(3)],
        [pltpu.VMEM((CHUNK, H), jnp.float32) for _ in range(3)],
        pltpu.VMEM_SHARED((N_NODES, H), jnp.float32),
        [pltpu.SemaphoreType.DMA for _ in range(3)],
        [pltpu.SemaphoreType.DMA for _ in range(3)],
        [pltpu.SemaphoreType.DMA for _ in range(6)],
        [pltpu.SemaphoreType.DMA for _ in range(3)],
        pltpu.SemaphoreType.DMA,
    ],
)
def _sc_aggregate(x_hbm, src_hbm, dst_hbm, ea_hbm, out_hbm,
                  svs, dvs, xrs, eas, acc_sh, gss, ess, iss, sss, zsem):
    c = lax.axis_index("c")
    s = lax.axis_index("s")
    wid = s * 2 + c

    zeros16 = jnp.zeros((L,), jnp.float32)
    zb = eas[0]

    def zrow(r, _):
        for j in range(H // L):
            zb[r, pl.ds(j * L, L)] = zeros16
        return 0

    lax.fori_loop(0, CHUNK, zrow, 0)
    slab = s * SLAB
    zcopies = [(zb, acc_sh.at[pl.ds(slab + i * CHUNK, CHUNK), :])
               for i in range(SLAB // CHUNK)]
    zcopies.append((zb.at[pl.ds(0, SLAB % CHUNK), :],
                    acc_sh.at[pl.ds(slab + (SLAB // CHUNK) * CHUNK,
                                    SLAB % CHUNK), :]))
    for zsrc, zdst in zcopies:
        pltpu.async_copy(zsrc, zdst, zsem)
    for zsrc, zdst in zcopies:
        pltpu.make_async_copy(zsrc, zdst, zsem).wait()

    @pl.when(s == 15)
    def _():
        pltpu.sync_copy(zb.at[pl.ds(0, 16), :],
                        acc_sh.at[pl.ds(16 * SLAB, 16), :])

    plsc.subcore_barrier()

    n_my = BASE_CHUNKS + jnp.where(wid < EXTRA, 1, 0)
    ebase = (BASE_CHUNKS * wid + jnp.minimum(wid, EXTRA)) * CHUNK

    def idx_start(kk, si):
        off = ebase + kk * CHUNK
        pltpu.async_copy(src_hbm.at[pl.ds(off, CHUNK)], svs[si], iss[si])
        pltpu.async_copy(dst_hbm.at[pl.ds(off, CHUNK)], dvs[si], iss[si])

    def idx_wait(si):
        pltpu.make_async_copy(src_hbm.at[pl.ds(0, CHUNK)], svs[si],
                              iss[si]).wait()
        pltpu.make_async_copy(dst_hbm.at[pl.ds(0, CHUNK)], dvs[si],
                              iss[si]).wait()

    def data_start(kk, b, si):
        pltpu.async_copy(x_hbm.at[svs[si]], xrs[b], gss[b])
        pltpu.async_copy(ea_hbm.at[pl.ds(ebase + kk * CHUNK, CHUNK), :],
                         eas[b], ess[b])

    def data_wait(b, si):
        pltpu.make_async_copy(x_hbm.at[svs[si]], xrs[b], gss[b]).wait()
        pltpu.make_async_copy(ea_hbm.at[pl.ds(0, CHUNK), :], eas[b],
                              ess[b]).wait()

    def compute(b):
        xr, ea = xrs[b], eas[b]

        @plsc.parallel_loop(0, CHUNK, unroll=4)
        def _(r):
            for j in range(H // L):
                sl = pl.ds(j * L, L)
                ea[r, sl] = jnp.maximum(xr[r, sl] + ea[r, sl], 0.0)

    def scatter_start(b, si):
        pltpu.async_copy(eas[b], acc_sh.at[dvs[si]], sss[b], add=True)

    def scatter_wait(b, si):
        pltpu.make_async_copy(eas[b], acc_sh.at[dvs[si]], sss[b]).wait()

    for kk in range(4):
        idx_start(kk, kk)
    idx_wait(0)
    data_start(0, 0, 0)

    @pl.loop(0, BASE_CHUNKS, step=6)
    def _(k):
        for u in range(6):
            kk = k + u
            b = u % 3
            si = u % 6
            bn = (u + 1) % 3
            sin = (u + 1) % 6
            bp = (u + 2) % 3
            sip = (u + 5) % 6

            @pl.when(kk + 1 < n_my)
            def _():
                idx_wait(sin)
                data_start(kk + 1, bn, sin)

            data_wait(b, si)
            compute(b)
            scatter_start(b, si)

            @pl.when(kk > 0)
            def _():
                scatter_wait(bp, sip)

            @pl.when(kk + 4 < n_my)
            def _():
                idx_start(kk + 4, (u + 4) % 6)

    @pl.when(wid < EXTRA)
    def _():
        data_wait(0, 0)
        compute(0)
        scatter_start(0, 0)
        scatter_wait(2, 5)
        scatter_wait(0, 0)

    @pl.when(wid >= EXTRA)
    def _():
        scatter_wait(2, 5)

    plsc.subcore_barrier()

    obase = c * N_NODES + slab
    wcopies = [(acc_sh.at[pl.ds(slab + i * CHUNK, CHUNK), :],
                out_hbm.at[pl.ds(obase + i * CHUNK, CHUNK), :])
               for i in range(SLAB // CHUNK)]
    wcopies.append((
        acc_sh.at[pl.ds(slab + (SLAB // CHUNK) * CHUNK, SLAB % CHUNK), :],
        out_hbm.at[pl.ds(obase + (SLAB // CHUNK) * CHUNK, SLAB % CHUNK), :]))
    for wsrc, wdst in wcopies:
        pltpu.async_copy(wsrc, wdst, zsem)
    for wsrc, wdst in wcopies:
        pltpu.make_async_copy(wsrc, wdst, zsem).wait()

    @pl.when(s == 15)
    def _():
        pltpu.sync_copy(acc_sh.at[pl.ds(16 * SLAB, 16), :],
                        out_hbm.at[pl.ds(c * N_NODES + 16 * SLAB, 16), :])


def _tc_layer_body(x_ref, p0_ref, p1_ref, wt_ref, b_ref, o_ref):
    h = x_ref[...] + p0_ref[...] + p1_ref[...]
    y = jnp.dot(h, wt_ref[...], preferred_element_type=jnp.float32) + b_ref[...]
    o_ref[...] = jnp.maximum(y, 0.0)


_TC_BLOCK = 1000


_P_BLOCKS = N_NODES // _TC_BLOCK


def _tc_layer(x, p, wt, b2d):
    grid = (N_NODES // _TC_BLOCK,)
    return pl.pallas_call(
        _tc_layer_body,
        grid=grid,
        in_specs=[
            pl.BlockSpec((_TC_BLOCK, H), lambda i: (i, 0)),
            pl.BlockSpec((_TC_BLOCK, H), lambda i: (i, 0)),
            pl.BlockSpec((_TC_BLOCK, H), lambda i: (i + _P_BLOCKS, 0)),
            pl.BlockSpec((H, H), lambda i: (0, 0)),
            pl.BlockSpec((1, H), lambda i: (0, 0)),
        ],
        out_specs=pl.BlockSpec((_TC_BLOCK, H), lambda i: (i, 0)),
        out_shape=jax.ShapeDtypeStruct((N_NODES, H), jnp.float32),
    )(x, p, p, wt, b2d)


def kernel(node_feats, edge_index, edge_attrs, W1, b1, W2, b2):
    src = edge_index[0].astype(jnp.int32)
    dst = edge_index[1].astype(jnp.int32)
    x = node_feats

    p = _sc_aggregate(x, src, dst, edge_attrs)
    x1 = _tc_layer(x, p, W1.T, b1.reshape(1, H))
    q = _sc_aggregate(x1, src, dst, edge_attrs)
    x2 = _tc_layer(x1, q, W2.T, b2.reshape(1, H))
    return x2
